# Initial kernel scaffold; baseline (speedup 1.0000x reference)
import jax
import jax.numpy as jnp
from jax.experimental import pallas as pl


def kernel(x, Wr, rms_g, W1, b1, W2, b2, fW1, fb1, fW2, fb2):
    raise NotImplementedError("write your pallas kernel here")


# speedup vs baseline: 1.9272x; 1.9272x over previous
import functools
import math

import jax
import jax.numpy as jnp
from jax import lax
from jax.experimental import pallas as pl
from jax.experimental.pallas import tpu as pltpu
from jax.experimental.pallas import tpu_sc as plsc

B, S, D = 2, 2048, 768
H = 3072
E = 8
K = 2
CAP_F = 1.25
FALLBACK_W = 1.0
N = B * S
C = int(math.ceil(CAP_F * K * N / E))
DUMP = E * C
R = E * C + C

BLK = 512
NB = N // BLK
HH = 768
HT = H // HH

NC = 2
NS = 16
NW = NC * NS
CH = 64


def _router_pos_body(x_ref, wr_ref, g_ref, pos_ref, ek_ref, vk_ref, carry_ref):
    s = pl.program_id(0)

    @pl.when(s == 0)
    def _():
        carry_ref[...] = jnp.zeros_like(carry_ref)

    k = s // NB
    xs = x_ref[...]
    ms = jnp.mean(xs * xs, axis=1, keepdims=True)
    xn = xs * lax.rsqrt(ms + 1e-6) * g_ref[...]
    logits = jnp.dot(xn, wr_ref[...], preferred_element_type=jnp.float32)
    gates = jax.nn.softmax(logits, axis=-1)

    ecols = lax.broadcasted_iota(jnp.int32, (BLK, E), 1)
    v0 = jnp.max(gates, axis=1, keepdims=True)
    e0 = jnp.min(jnp.where(gates == v0, ecols, E), axis=1, keepdims=True)
    gm = jnp.where(ecols == e0, -1.0, gates)
    v1 = jnp.max(gm, axis=1, keepdims=True)
    e1 = jnp.min(jnp.where(gm == v1, ecols, E), axis=1, keepdims=True)

    ek = jnp.where(k == 0, e0, e1)
    vk = jnp.where(k == 0, v0, v1)
    onehot = (ecols == ek).astype(jnp.float32)

    ri = lax.broadcasted_iota(jnp.int32, (BLK, BLK), 0)
    ci = lax.broadcasted_iota(jnp.int32, (BLK, BLK), 1)
    tri = (ci < ri).astype(jnp.float32)
    cum = jnp.dot(tri, onehot, preferred_element_type=jnp.float32)

    carry = carry_ref[0:1, 0:E]
    posv = jnp.sum((carry + cum) * onehot, axis=1)
    pos_ref[0, 0, :] = posv
    ek_ref[0, 0, :] = ek[:, 0]
    vk_ref[0, 0, :] = vk[:, 0]
    carry_ref[0:1, 0:E] = carry + jnp.sum(onehot, axis=0, keepdims=True)


def _router_positions(x_flat, Wr, g2):
    out3 = (K * NB, 1, BLK)
    return pl.pallas_call(
        _router_pos_body,
        grid=(K * NB,),
        in_specs=[
            pl.BlockSpec((BLK, D), lambda s: (s % NB, 0)),
            pl.BlockSpec((D, E), lambda s: (0, 0)),
            pl.BlockSpec((1, D), lambda s: (0, 0)),
        ],
        out_specs=[
            pl.BlockSpec((1, 1, BLK), lambda s: (s, 0, 0)),
            pl.BlockSpec((1, 1, BLK), lambda s: (s, 0, 0)),
            pl.BlockSpec((1, 1, BLK), lambda s: (s, 0, 0)),
        ],
        out_shape=[
            jax.ShapeDtypeStruct(out3, jnp.float32),
            jax.ShapeDtypeStruct(out3, jnp.int32),
            jax.ShapeDtypeStruct(out3, jnp.float32),
        ],
        scratch_shapes=[pltpu.VMEM((8, 128), jnp.float32)],
        compiler_params=pltpu.CompilerParams(
            dimension_semantics=("arbitrary",)),
    )(x_flat, Wr, g2)


def _weights_body(p0_ref, p1_ref, e0_ref, e1_ref, v0_ref, v1_ref,
                  w0_ref, w1_ref, dd0_ref, dd1_ref, dg0_ref, dg1_ref,
                  drop_ref):
    p0 = p0_ref[...]
    p1 = p1_ref[...]
    e0 = e0_ref[...]
    e1 = e1_ref[...]
    v0 = v0_ref[...]
    v1 = v1_ref[...]
    k0 = p0 < float(C)
    k1 = p1 < float(C)
    w0 = jnp.where(k0, v0, 0.0)
    w1 = jnp.where(k1, v1, 0.0)
    den = jnp.maximum(w0 + w1, 1e-9)
    w0_ref[...] = w0 / den
    w1_ref[...] = w1 / den
    s0 = e0 * C + p0.astype(jnp.int32)
    s1 = e1 * C + p1.astype(jnp.int32)
    dd0_ref[...] = jnp.where(k0, s0, DUMP)
    dd1_ref[...] = jnp.where(k1, s1, DUMP)
    dg0_ref[...] = jnp.where(k0, s0, 0)
    dg1_ref[...] = jnp.where(k1, s1, 0)
    drop_ref[...] = jnp.logical_and(~k0, ~k1).astype(jnp.float32)


def _weights(pos3, ek3, vk3):
    blk = pl.BlockSpec((1, 1, BLK), lambda b: (b, 0, 0))
    blk_hi = pl.BlockSpec((1, 1, BLK), lambda b: (b + NB, 0, 0))
    out3 = (NB, 1, BLK)
    return pl.pallas_call(
        _weights_body,
        grid=(NB,),
        in_specs=[blk, blk_hi, blk, blk_hi, blk, blk_hi],
        out_specs=[blk] * 7,
        out_shape=[
            jax.ShapeDtypeStruct(out3, jnp.float32),
            jax.ShapeDtypeStruct(out3, jnp.float32),
            jax.ShapeDtypeStruct(out3, jnp.int32),
            jax.ShapeDtypeStruct(out3, jnp.int32),
            jax.ShapeDtypeStruct(out3, jnp.int32),
            jax.ShapeDtypeStruct(out3, jnp.int32),
            jax.ShapeDtypeStruct(out3, jnp.float32),
        ],
    )(pos3, pos3, ek3, ek3, vk3, vk3)


def _sc_dispatch(x_flat, dd):
    mesh = plsc.VectorSubcoreMesh(core_axis_name="c", subcore_axis_name="s")

    @functools.partial(
        pl.kernel,
        out_type=jax.ShapeDtypeStruct((R, D), jnp.float32),
        mesh=mesh,
        scratch_types=[
            pltpu.VMEM((CH,), jnp.int32),
            pltpu.VMEM((CH, D), jnp.float32),
            pltpu.SemaphoreType.DMA,
        ],
    )
    def disp(x_hbm, dd_hbm, xec_hbm, idx_v, rows_v, sem):
        wid = lax.axis_index("s") * NC + lax.axis_index("c")
        per_w = N // NW
        for k in range(K):
            for c in range(per_w // CH):
                base = wid * per_w + c * CH
                pltpu.sync_copy(dd_hbm.at[k, pl.ds(base, CH)], idx_v)
                pltpu.sync_copy(x_hbm.at[pl.ds(base, CH)], rows_v)
                pltpu.async_copy(rows_v, xec_hbm.at[idx_v], sem).wait()

    return disp(x_flat, dd)


def _ffn_body(xec_ref, w1a_ref, w1b_ref, b1a_ref, b1b_ref, w2_ref, b2_ref,
              y_ref):
    t = pl.program_id(1)
    xb = xec_ref[...].astype(jnp.bfloat16)
    a = jnp.dot(xb, w1a_ref[0], preferred_element_type=jnp.float32)
    a = a + b1a_ref[0]
    bb = jnp.dot(xb, w1b_ref[0], preferred_element_type=jnp.float32)
    bb = bb + b1b_ref[0]
    g = (a * jax.nn.sigmoid(a)) * bb
    part = jnp.dot(g.astype(jnp.bfloat16), w2_ref[0],
                   preferred_element_type=jnp.float32)

    @pl.when(t == 0)
    def _():
        y_ref[...] = part + b2_ref[0]

    @pl.when(t > 0)
    def _():
        y_ref[...] += part


def _ffn(x_ec, W1b, b1, W2b, b2):
    return pl.pallas_call(
        _ffn_body,
        grid=(E, HT),
        in_specs=[
            pl.BlockSpec((C, D), lambda e, t: (e, 0)),
            pl.BlockSpec((1, D, HH), lambda e, t: (e, 0, t)),
            pl.BlockSpec((1, D, HH), lambda e, t: (e, 0, HT + t)),
            pl.BlockSpec((1, 1, HH), lambda e, t: (e, 0, t)),
            pl.BlockSpec((1, 1, HH), lambda e, t: (e, 0, HT + t)),
            pl.BlockSpec((1, HH, D), lambda e, t: (e, t, 0)),
            pl.BlockSpec((1, 1, D), lambda e, t: (e, 0, 0)),
        ],
        out_specs=pl.BlockSpec((C, D), lambda e, t: (e, 0)),
        out_shape=jax.ShapeDtypeStruct((E * C, D), jnp.float32),
        compiler_params=pltpu.CompilerParams(
            dimension_semantics=("arbitrary", "arbitrary")),
    )(x_ec, W1b, W1b, b1.reshape(E, 1, 2 * H), b1.reshape(E, 1, 2 * H),
      W2b, b2.reshape(E, 1, D))


def _sc_gather(y_ec, dg):
    mesh = plsc.VectorSubcoreMesh(core_axis_name="c", subcore_axis_name="s")

    @functools.partial(
        pl.kernel,
        out_type=jax.ShapeDtypeStruct((K, N, D), jnp.float32),
        mesh=mesh,
        scratch_types=[
            pltpu.VMEM((CH,), jnp.int32),
            pltpu.VMEM((CH, D), jnp.float32),
            pltpu.SemaphoreType.DMA,
        ],
    )
    def gath(yec_hbm, dg_hbm, yg_hbm, idx_v, rows_v, sem):
        wid = lax.axis_index("s") * NC + lax.axis_index("c")
        per_w = N // NW
        for k in range(K):
            for c in range(per_w // CH):
                base = wid * per_w + c * CH
                pltpu.sync_copy(dg_hbm.at[k, pl.ds(base, CH)], idx_v)
                pltpu.async_copy(yec_hbm.at[idx_v], rows_v, sem).wait()
                pltpu.sync_copy(rows_v, yg_hbm.at[k, pl.ds(base, CH)])

    return gath(y_ec, dg)


def _final_body(y0_ref, y1_ref, w0_ref, w1_ref, drop_ref, x_ref,
                fw1_ref, fw2_ref, fb1_ref, fb2_ref, out_ref,
                wa_ref, wb_ref, w2s_ref, s0, s1, s2):
    w0 = w0_ref[...]
    w1 = w1_ref[...]
    y0 = y0_ref[...]
    y1 = y1_ref[...]
    out_ref[...] = (jnp.where(w0 != 0.0, w0 * y0, 0.0)
                    + jnp.where(w1 != 0.0, w1 * y1, 0.0))

    dropv = drop_ref[...]

    @pl.when(jnp.sum(dropv) > 0.0)
    def _():
        xb = x_ref[...].astype(jnp.bfloat16)
        acc = jnp.zeros((BLK, D), jnp.float32)
        for t in range(HT):
            c1 = pltpu.make_async_copy(
                fw1_ref.at[:, pl.ds(t * HH, HH)], wa_ref, s0)
            c2 = pltpu.make_async_copy(
                fw1_ref.at[:, pl.ds(H + t * HH, HH)], wb_ref, s1)
            c3 = pltpu.make_async_copy(
                fw2_ref.at[pl.ds(t * HH, HH), :], w2s_ref, s2)
            c1.start()
            c2.start()
            c3.start()
            c1.wait()
            c2.wait()
            c3.wait()
            a = jnp.dot(xb, wa_ref[...], preferred_element_type=jnp.float32)
            a = a + fb1_ref[0:1, t * HH:(t + 1) * HH]
            bb = jnp.dot(xb, wb_ref[...], preferred_element_type=jnp.float32)
            bb = bb + fb1_ref[0:1, H + t * HH:H + (t + 1) * HH]
            g = (a * jax.nn.sigmoid(a)) * bb
            acc = acc + jnp.dot(g.astype(jnp.bfloat16), w2s_ref[...],
                                preferred_element_type=jnp.float32)
        acc = acc + fb2_ref[...]
        out_ref[...] += jnp.where(dropv != 0.0, FALLBACK_W * acc, 0.0)


def _final(y_g2, w0c, w1c, dropc, x_flat, fW1b, fW2b, fb1_2, fb2_2):
    return pl.pallas_call(
        _final_body,
        grid=(NB,),
        in_specs=[
            pl.BlockSpec((BLK, D), lambda b: (b, 0)),
            pl.BlockSpec((BLK, D), lambda b: (b + NB, 0)),
            pl.BlockSpec((BLK, 1), lambda b: (b, 0)),
            pl.BlockSpec((BLK, 1), lambda b: (b, 0)),
            pl.BlockSpec((BLK, 1), lambda b: (b, 0)),
            pl.BlockSpec((BLK, D), lambda b: (b, 0)),
            pl.BlockSpec(memory_space=pl.ANY),
            pl.BlockSpec(memory_space=pl.ANY),
            pl.BlockSpec((1, 2 * H), lambda b: (0, 0)),
            pl.BlockSpec((1, D), lambda b: (0, 0)),
        ],
        out_specs=pl.BlockSpec((BLK, D), lambda b: (b, 0)),
        out_shape=jax.ShapeDtypeStruct((N, D), jnp.float32),
        scratch_shapes=[
            pltpu.VMEM((D, HH), jnp.bfloat16),
            pltpu.VMEM((D, HH), jnp.bfloat16),
            pltpu.VMEM((HH, D), jnp.bfloat16),
            pltpu.SemaphoreType.DMA,
            pltpu.SemaphoreType.DMA,
            pltpu.SemaphoreType.DMA,
        ],
    )(y_g2, y_g2, w0c, w1c, dropc, x_flat, fW1b, fW2b, fb1_2, fb2_2)


def kernel(x, Wr, rms_g, W1, b1, W2, b2, fW1, fb1, fW2, fb2):
    x_flat = x.reshape(N, D)
    g2 = rms_g.reshape(1, D)

    pos3, ek3, vk3 = _router_positions(x_flat, Wr, g2)
    w0b, w1b, dd0, dd1, dg0, dg1, dropb = _weights(pos3, ek3, vk3)

    dd = jnp.concatenate([dd0.reshape(1, N), dd1.reshape(1, N)], axis=0)
    dg = jnp.concatenate([dg0.reshape(1, N), dg1.reshape(1, N)], axis=0)

    x_ec = _sc_dispatch(x_flat, dd)

    W1b = W1.astype(jnp.bfloat16)
    W2b = W2.astype(jnp.bfloat16)
    y_ec = _ffn(x_ec, W1b, b1, W2b, b2)

    y_g = _sc_gather(y_ec, dg)

    out = _final(
        y_g.reshape(K * N, D),
        w0b.reshape(N, 1),
        w1b.reshape(N, 1),
        dropb.reshape(N, 1),
        x_flat,
        fW1.astype(jnp.bfloat16),
        fW2.astype(jnp.bfloat16),
        fb1.reshape(1, 2 * H),
        fb2.reshape(1, D),
    )
    return out.reshape(B, S, D)
